# baseline (speedup 1.0000x reference)
import functools

import jax
import jax.numpy as jnp
from jax import lax
from jax.experimental import pallas as pl
from jax.experimental.pallas import tpu as pltpu
from jax.experimental.pallas import tpu_sc as plsc

_NC = 2
_NS = 16
_NW = _NC * _NS

_C = 128


def _sc_gather_sums(features, sidx_t, idx_t, n_self_rows, n_sum_rows, S):
    d_feat = features.shape[1]
    cpt = n_sum_rows // (_NW * _C)
    s_cpt = n_self_rows // (_NW * _C)
    assert n_sum_rows % (_NW * _C) == 0 and n_self_rows % (_NW * _C) == 0
    assert S % 2 == 0
    rpc = S // 2
    n_rounds = cpt * rpc
    n_pairs = cpt // 2
    tail = cpt % 2
    assert rpc % 2 == 1
    n_vregs = d_feat // 16
    r2 = 2 * _C

    mesh = plsc.VectorSubcoreMesh(core_axis_name="c", subcore_axis_name="s")

    @functools.partial(
        pl.kernel,
        out_type=(
            jax.ShapeDtypeStruct((n_self_rows, d_feat), jnp.float32),
            jax.ShapeDtypeStruct((n_sum_rows, d_feat), jnp.float32),
        ),
        mesh=mesh,
        scratch_types=[
            pltpu.VMEM((cpt * S * _C,), jnp.int32),
            pltpu.VMEM((s_cpt * _C,), jnp.int32),
            pltpu.VMEM((r2, d_feat), jnp.float32),
            pltpu.VMEM((r2, d_feat), jnp.float32),
            pltpu.VMEM((_C, d_feat), jnp.float32),
            pltpu.VMEM((_C, d_feat), jnp.float32),
            pltpu.SemaphoreType.DMA,
            pltpu.SemaphoreType.DMA,
        ],
    )
    def sc_body(sidx_hbm, idx_hbm, feat_hbm, self_out, nsum_out,
                idx_v, sidx_v, b0, b1, a0, a1, sem0, sem1):
        wid = lax.axis_index("s") * _NC + lax.axis_index("c")
        bufs = (b0, b1)
        sems = (sem0, sem1)

        pltpu.sync_copy(idx_hbm.at[wid], idx_v)
        pltpu.sync_copy(sidx_hbm.at[wid], sidx_v)

        def issue(rnd, parity):
            pltpu.async_copy(
                feat_hbm.at[idx_v.at[pl.ds(rnd * r2, r2)]],
                bufs[parity], sems[parity])

        def wait_ring(parity):
            pltpu.make_async_copy(
                feat_hbm.at[pl.ds(0, r2)], bufs[parity], sems[parity]).wait()

        def accumulate(acc, buf, init):
            def body(j, carry):
                for k in range(n_vregs):
                    sl = pl.ds(k * 16, 16)
                    v = buf[j, sl] + buf[_C + j, sl]
                    if init:
                        acc[j, sl] = v
                    else:
                        acc[j, sl] = acc[j, sl] + v
                return carry

            lax.fori_loop(0, _C, body, 0)

        issue(0, 0)

        def pair_body(u, carry):
            base_r = u * 2 * rpc
            base_c = wid * cpt + u * 2
            for r in range(2 * rpc):
                wait_ring(r % 2)
                issue(base_r + r + 1, (r + 1) % 2)
                acc = a0 if r < rpc else a1
                accumulate(acc, bufs[r % 2], r % rpc == 0)
                if r == rpc - 1:
                    pltpu.sync_copy(a0, nsum_out.at[pl.ds(base_c * _C, _C)])
                if r == 2 * rpc - 1:
                    pltpu.sync_copy(
                        a1, nsum_out.at[pl.ds((base_c + 1) * _C, _C)])
            return carry

        lax.fori_loop(0, n_pairs, pair_body, 0)

        if tail:
            base_r = n_pairs * 2 * rpc
            base_c = wid * cpt + n_pairs * 2
            for r in range(rpc):
                wait_ring(r % 2)
                if r < rpc - 1:
                    issue(base_r + r + 1, (r + 1) % 2)
                accumulate(a0, bufs[r % 2], r == 0)
            pltpu.sync_copy(a0, nsum_out.at[pl.ds(base_c * _C, _C)])

        handles = [None] * s_cpt
        handles[0] = pltpu.async_copy(
            feat_hbm.at[sidx_v.at[pl.ds(0, _C)]],
            bufs[0].at[pl.ds(0, _C)], sems[0])
        for q in range(s_cpt):
            handles[q].wait()
            if q + 1 < s_cpt:
                handles[q + 1] = pltpu.async_copy(
                    feat_hbm.at[sidx_v.at[pl.ds((q + 1) * _C, _C)]],
                    bufs[(q + 1) % 2].at[pl.ds(0, _C)], sems[(q + 1) % 2])
            pltpu.sync_copy(
                bufs[q % 2].at[pl.ds(0, _C)],
                self_out.at[pl.ds((wid * s_cpt + q) * _C, _C)])

    return sc_body(sidx_t, idx_t, features)


def _tc_combine(self_3, nsum_3, w_cat, side, B, rb):
    d_feat = self_3.shape[2]
    D = w_cat.shape[0]
    out0 = w_cat.shape[2]

    def body(self_ref, nsum_ref, w_ref, out_ref):
        comb = jnp.concatenate([self_ref[0], nsum_ref[0]], axis=-1)
        h = jax.lax.dot_general(comb, w_ref[0], (((1,), (0,)), ((), ())),
                                preferred_element_type=jnp.float32)
        out_ref[...] = jnp.maximum(h, 0.0)

    return pl.pallas_call(
        body,
        grid=(B // rb, D),
        in_specs=[
            pl.BlockSpec((1, rb, d_feat), lambda i, d: (side, i, 0)),
            pl.BlockSpec((1, rb, d_feat), lambda i, d: (side * D + d, i, 0)),
            pl.BlockSpec((1, 2 * d_feat, out0), lambda i, d: (d, 0, 0)),
        ],
        out_specs=pl.BlockSpec((rb, out0), lambda i, d: (i, d)),
        out_shape=jax.ShapeDtypeStruct((B, D * out0), jnp.float32),
    )(self_3, nsum_3, w_cat)


def kernel(features, dims, counts, source_nodes, source_to_neighs_dims,
           target_nodes, target_to_neighs_dims, num_samples, W_dims):
    del dims, counts
    d_feat = features.shape[1]
    D, B, S = source_to_neighs_dims.shape
    out0 = W_dims.shape[2]

    bp_unit = _NW * _C // 2
    Bp = ((B + bp_unit - 1) // bp_unit) * bp_unit
    n_sum_rows = 2 * D * Bp
    n_self_rows = 2 * Bp

    neigh_cat = jnp.concatenate(
        [source_to_neighs_dims, target_to_neighs_dims], axis=0)
    neigh_cat = jnp.pad(neigh_cat, ((0, 0), (0, Bp - B), (0, 0)))
    idx_t = neigh_cat.reshape(n_sum_rows // _C, _C, S).transpose(0, 2, 1)
    idx_t = idx_t.reshape(_NW, (n_sum_rows // (_NW * _C)) * S * _C)

    self_cat = jnp.concatenate(
        [jnp.pad(source_nodes, (0, Bp - B)),
         jnp.pad(target_nodes, (0, Bp - B))])
    sidx_t = self_cat.reshape(_NW, n_self_rows // _NW)

    self_rows, neigh_sum = _sc_gather_sums(
        features, sidx_t, idx_t, n_self_rows, n_sum_rows, S)

    inv_n = 1.0 / jnp.asarray(num_samples, jnp.float32)
    w_cat = jnp.concatenate(
        [W_dims[:, :d_feat, :], W_dims[:, d_feat:, :] * inv_n], axis=1)

    self_3 = self_rows.reshape(2, Bp, d_feat)
    nsum_3 = neigh_sum.reshape(2 * D, Bp, d_feat)

    rb = 2000
    assert B % rb == 0
    x_sources = _tc_combine(self_3, nsum_3, w_cat, 0, B, rb)
    x_targets = _tc_combine(self_3, nsum_3, w_cat, 1, B, rb)
    return (x_sources, x_targets)


# speedup vs baseline: 1.0163x; 1.0163x over previous
import functools

import jax
import jax.numpy as jnp
from jax import lax
from jax.experimental import pallas as pl
from jax.experimental.pallas import tpu as pltpu
from jax.experimental.pallas import tpu_sc as plsc

_NC = 2
_NS = 16
_NW = _NC * _NS

_C = 128


def _sc_gather_sums(features, sidx_t, idx_t, n_self_rows, n_sum_rows, S):
    d_feat = features.shape[1]
    cpt = n_sum_rows // (_NW * _C)
    s_cpt = n_self_rows // (_NW * _C)
    assert n_sum_rows % (_NW * _C) == 0 and n_self_rows % (_NW * _C) == 0
    assert S % 2 == 0
    rpc = S // 2
    n_rounds = cpt * rpc
    n_pairs = cpt // 2
    tail = cpt % 2
    assert rpc % 2 == 1
    n_vregs = d_feat // 16
    r2 = 2 * _C

    mesh = plsc.VectorSubcoreMesh(core_axis_name="c", subcore_axis_name="s")

    @functools.partial(
        pl.kernel,
        out_type=(
            jax.ShapeDtypeStruct((n_self_rows, d_feat), jnp.float32),
            jax.ShapeDtypeStruct((n_sum_rows, d_feat), jnp.float32),
        ),
        mesh=mesh,
        scratch_types=[
            pltpu.VMEM((cpt * S * _C,), jnp.int32),
            pltpu.VMEM((s_cpt * _C,), jnp.int32),
            pltpu.VMEM((r2, d_feat), jnp.float32),
            pltpu.VMEM((r2, d_feat), jnp.float32),
            pltpu.VMEM((_C, d_feat), jnp.float32),
            pltpu.VMEM((_C, d_feat), jnp.float32),
            pltpu.SemaphoreType.DMA,
            pltpu.SemaphoreType.DMA,
            pltpu.SemaphoreType.DMA,
            pltpu.SemaphoreType.DMA,
        ],
    )
    def sc_body(sidx_hbm, idx_hbm, feat_hbm, self_out, nsum_out,
                idx_v, sidx_v, b0, b1, a0, a1, sem0, sem1, osem0, osem1):
        wid = lax.axis_index("s") * _NC + lax.axis_index("c")
        bufs = (b0, b1)
        sems = (sem0, sem1)

        pltpu.sync_copy(idx_hbm.at[wid], idx_v)
        pltpu.sync_copy(sidx_hbm.at[wid], sidx_v)

        def issue(rnd, parity):
            pltpu.async_copy(
                feat_hbm.at[idx_v.at[pl.ds(rnd * r2, r2)]],
                bufs[parity], sems[parity])

        def wait_ring(parity):
            pltpu.make_async_copy(
                feat_hbm.at[pl.ds(0, r2)], bufs[parity], sems[parity]).wait()

        def drain_acc(acc, osem):
            pltpu.make_async_copy(
                feat_hbm.at[pl.ds(0, _C)], acc, osem).wait()

        def accumulate(acc, buf, init):
            def body(j, carry):
                for k in range(n_vregs):
                    sl = pl.ds(k * 16, 16)
                    v = buf[j, sl] + buf[_C + j, sl]
                    if init:
                        acc[j, sl] = v
                    else:
                        acc[j, sl] = acc[j, sl] + v
                return carry

            lax.fori_loop(0, _C, body, 0)

        issue(0, 0)

        def pair_body(u, carry):
            base_r = u * 2 * rpc
            base_c = wid * cpt + u * 2
            for r in range(2 * rpc):
                wait_ring(r % 2)
                issue(base_r + r + 1, (r + 1) % 2)
                if r == 0:
                    pl.when(u > 0)(lambda: drain_acc(a0, osem0))
                if r == rpc:
                    pl.when(u > 0)(lambda: drain_acc(a1, osem1))
                acc = a0 if r < rpc else a1
                accumulate(acc, bufs[r % 2], r % rpc == 0)
                if r == rpc - 1:
                    pltpu.async_copy(
                        a0, nsum_out.at[pl.ds(base_c * _C, _C)], osem0)
                if r == 2 * rpc - 1:
                    pltpu.async_copy(
                        a1, nsum_out.at[pl.ds((base_c + 1) * _C, _C)], osem1)
            return carry

        lax.fori_loop(0, n_pairs, pair_body, 0)

        if tail:
            base_r = n_pairs * 2 * rpc
            base_c = wid * cpt + n_pairs * 2
            for r in range(rpc):
                wait_ring(r % 2)
                if r < rpc - 1:
                    issue(base_r + r + 1, (r + 1) % 2)
                if r == 0 and n_pairs > 0:
                    drain_acc(a0, osem0)
                accumulate(a0, bufs[r % 2], r == 0)
            pltpu.async_copy(a0, nsum_out.at[pl.ds(base_c * _C, _C)], osem0)

        handles = [None] * s_cpt
        handles[0] = pltpu.async_copy(
            feat_hbm.at[sidx_v.at[pl.ds(0, _C)]],
            bufs[0].at[pl.ds(0, _C)], sems[0])
        for q in range(s_cpt):
            handles[q].wait()
            if q + 1 < s_cpt:
                handles[q + 1] = pltpu.async_copy(
                    feat_hbm.at[sidx_v.at[pl.ds((q + 1) * _C, _C)]],
                    bufs[(q + 1) % 2].at[pl.ds(0, _C)], sems[(q + 1) % 2])
            pltpu.sync_copy(
                bufs[q % 2].at[pl.ds(0, _C)],
                self_out.at[pl.ds((wid * s_cpt + q) * _C, _C)])

        if tail or n_pairs > 0:
            drain_acc(a0, osem0)
        if n_pairs > 0:
            drain_acc(a1, osem1)

    return sc_body(sidx_t, idx_t, features)


def _tc_combine(self_3, nsum_3, w_cat, B, rb):
    d_feat = self_3.shape[2]
    D = w_cat.shape[0]
    out0 = w_cat.shape[2]

    def body(self_ref, nsum_a, nsum_b, w_ref, out_a, out_b):
        comb0 = jnp.concatenate([self_ref[0], nsum_a[0]], axis=-1)
        comb1 = jnp.concatenate([self_ref[1], nsum_b[0]], axis=-1)
        dn = (((1,), (0,)), ((), ()))
        out_a[...] = jnp.maximum(jax.lax.dot_general(
            comb0, w_ref[0], dn, preferred_element_type=jnp.float32), 0.0)
        out_b[...] = jnp.maximum(jax.lax.dot_general(
            comb1, w_ref[0], dn, preferred_element_type=jnp.float32), 0.0)

    return pl.pallas_call(
        body,
        grid=(B // rb, D),
        in_specs=[
            pl.BlockSpec((2, rb, d_feat), lambda i, d: (0, i, 0)),
            pl.BlockSpec((1, rb, d_feat), lambda i, d: (d, i, 0)),
            pl.BlockSpec((1, rb, d_feat), lambda i, d: (D + d, i, 0)),
            pl.BlockSpec((1, 2 * d_feat, out0), lambda i, d: (d, 0, 0)),
        ],
        out_specs=[
            pl.BlockSpec((rb, out0), lambda i, d: (i, d)),
            pl.BlockSpec((rb, out0), lambda i, d: (i, d)),
        ],
        out_shape=[
            jax.ShapeDtypeStruct((B, D * out0), jnp.float32),
            jax.ShapeDtypeStruct((B, D * out0), jnp.float32),
        ],
    )(self_3, nsum_3, nsum_3, w_cat)


def kernel(features, dims, counts, source_nodes, source_to_neighs_dims,
           target_nodes, target_to_neighs_dims, num_samples, W_dims):
    del dims, counts
    d_feat = features.shape[1]
    D, B, S = source_to_neighs_dims.shape
    out0 = W_dims.shape[2]

    bp_unit = _NW * _C // 2
    Bp = ((B + bp_unit - 1) // bp_unit) * bp_unit
    n_sum_rows = 2 * D * Bp
    n_self_rows = 2 * Bp

    neigh_cat = jnp.concatenate(
        [source_to_neighs_dims, target_to_neighs_dims], axis=0)
    neigh_cat = jnp.pad(neigh_cat, ((0, 0), (0, Bp - B), (0, 0)))
    idx_t = neigh_cat.reshape(n_sum_rows // _C, _C, S).transpose(0, 2, 1)
    idx_t = idx_t.reshape(_NW, (n_sum_rows // (_NW * _C)) * S * _C)

    self_cat = jnp.concatenate(
        [jnp.pad(source_nodes, (0, Bp - B)),
         jnp.pad(target_nodes, (0, Bp - B))])
    sidx_t = self_cat.reshape(_NW, n_self_rows // _NW)

    self_rows, neigh_sum = _sc_gather_sums(
        features, sidx_t, idx_t, n_self_rows, n_sum_rows, S)

    inv_n = 1.0 / jnp.asarray(num_samples, jnp.float32)
    w_cat = jnp.concatenate(
        [W_dims[:, :d_feat, :], W_dims[:, d_feat:, :] * inv_n], axis=1)

    self_3 = self_rows.reshape(2, Bp, d_feat)
    nsum_3 = neigh_sum.reshape(2 * D, Bp, d_feat)

    rb = 2000
    assert B % rb == 0
    x_sources, x_targets = _tc_combine(self_3, nsum_3, w_cat, B, rb)
    return (x_sources, x_targets)
